# baseline (speedup 1.0000x reference)
import functools

import jax
import jax.numpy as jnp
from jax import lax
from jax.experimental import pallas as pl
from jax.experimental.pallas import tpu as pltpu
from jax.experimental.pallas import tpu_sc as plsc

_N_EMB = 1024
_DIM = 64
_ROWS = 9216
_HALF_SLABS = 8
_HALF_ROWS = 4608
_TILE = 1024
_IDX_ROWS = _HALF_ROWS // 128

_NW = 32
_B_PER_W = _HALF_ROWS // _NW
_CHUNK = 72
_N_CHUNK = _B_PER_W // _CHUNK


def _tile_sizes():
    sizes = []
    left = _HALF_ROWS
    while left > 0:
        t = min(_TILE, left)
        sizes.append(t)
        left -= t
    return sizes


def _vq_tc_body(build_table, x_ref, emb_ref, idx_ref, loss_ref, *rest):
    emb = emb_ref[...]
    emb_t = emb.T
    b = jnp.sum(emb_t * emb_t, axis=0, keepdims=True)
    xt_all = x_ref[...].reshape(_HALF_ROWS, _DIM)
    loss = jnp.zeros((1, 1), jnp.float32)
    r0 = 0
    for t in _tile_sizes():
        xt = lax.slice(xt_all, (r0, 0), (r0 + t, _DIM))
        a = jnp.sum(xt * xt, axis=1, keepdims=True)
        c2 = lax.dot_general(xt + xt, emb_t, (((1,), (0,)), ((), ())),
                             preferred_element_type=jnp.float32)
        d = (a + b) - c2
        m = jnp.min(d, axis=1, keepdims=True)
        iota = lax.broadcasted_iota(jnp.int32, d.shape, 1)
        cand = jnp.where(d == m, iota, _N_EMB)
        s = [lax.slice(cand, (0, 128 * k), (t, 128 * (k + 1)))
             for k in range(8)]
        p = jnp.minimum(jnp.minimum(jnp.minimum(s[0], s[1]),
                                    jnp.minimum(s[2], s[3])),
                        jnp.minimum(jnp.minimum(s[4], s[5]),
                                    jnp.minimum(s[6], s[7])))
        idx_row = jnp.min(p.T, axis=0, keepdims=True)
        idx_ref[pl.ds(r0 // 128, t // 128), :] = idx_row.reshape(t // 128, 128)
        loss = loss + jnp.sum(m, keepdims=True)
        r0 += t
    loss_ref[...] = loss

    if build_table:
        emb_c_ref = rest[0]
        rr = lax.broadcasted_iota(jnp.int32, (_N_EMB // 2, _N_EMB), 0)
        cc = lax.broadcasted_iota(jnp.int32, (_N_EMB // 2, _N_EMB), 1)
        s_even = (cc == rr + rr).astype(jnp.float32)
        s_odd = (cc == rr + rr + 1).astype(jnp.float32)
        dn = (((1,), (0,)), ((), ()))
        even = lax.dot_general(s_even, emb, dn,
                               preferred_element_type=jnp.float32)
        odd = lax.dot_general(s_odd, emb, dn,
                              preferred_element_type=jnp.float32)
        emb_c_ref[...] = jnp.concatenate([even, odd], axis=1)


def _distances_argmin_loss(x, emb, half, build_table):
    out_specs = [
        pl.BlockSpec((_IDX_ROWS, 128), lambda i: (0, 0)),
        pl.BlockSpec((1, 1), lambda i: (0, 0)),
    ]
    out_shape = [
        jax.ShapeDtypeStruct((_IDX_ROWS, 128), jnp.int32),
        jax.ShapeDtypeStruct((1, 1), jnp.float32),
    ]
    if build_table:
        out_specs.append(pl.BlockSpec((_N_EMB // 2, 2 * _DIM),
                                      lambda i: (0, 0)))
        out_shape.append(jax.ShapeDtypeStruct((_N_EMB // 2, 2 * _DIM),
                                              jnp.float32))
    return pl.pallas_call(
        functools.partial(_vq_tc_body, build_table),
        grid=(1,),
        in_specs=[
            pl.BlockSpec((_HALF_SLABS, 576, _DIM),
                         lambda i, h=half: (h, 0, 0)),
            pl.BlockSpec((_N_EMB, _DIM), lambda i: (0, 0)),
        ],
        out_specs=out_specs,
        out_shape=out_shape,
    )(x, emb)


@functools.cache
def _make_sc_gather():
    @functools.partial(
        pl.kernel,
        mesh=plsc.VectorSubcoreMesh(core_axis_name="c", subcore_axis_name="s"),
        out_type=jax.ShapeDtypeStruct((_HALF_SLABS, 576, _DIM), jnp.float32),
        scratch_types=[
            pltpu.VMEM((_N_CHUNK, _CHUNK), jnp.int32),
            pltpu.VMEM((_B_PER_W, _DIM), jnp.float32),
            pltpu.SemaphoreType.DMA,
        ],
        compiler_params=pltpu.CompilerParams(use_tc_tiling_on_sc=False),
    )
    def _sc_gather(table_hbm, idx_hbm, out_hbm, idx_v, rows_v, sem):
        wid = lax.axis_index("s") * 2 + lax.axis_index("c")
        pltpu.sync_copy(idx_hbm.at[wid], idx_v)
        copies = [
            pltpu.async_copy(table_hbm.at[idx_v.at[c]],
                             rows_v.at[pl.ds(c * _CHUNK, _CHUNK)], sem)
            for c in range(_N_CHUNK)
        ]
        for cp in copies:
            cp.wait()
        pltpu.sync_copy(
            rows_v,
            out_hbm.at[wid // 4, pl.ds((wid % 4) * _B_PER_W, _B_PER_W)])

    return _sc_gather


def kernel(x, emb_weight):
    idx_a, loss_a, emb_c = _distances_argmin_loss(x, emb_weight, 0, True)
    idx_b, loss_b = _distances_argmin_loss(x, emb_weight, 1, False)
    table = emb_c.reshape(_N_EMB, _DIM)
    gather = _make_sc_gather()
    q_a = gather(table, idx_a.reshape(_NW, _N_CHUNK, _CHUNK))
    q_b = gather(table, idx_b.reshape(_NW, _N_CHUNK, _CHUNK))
    quantized = jnp.concatenate([q_a, q_b], axis=0)
    vq_loss = (loss_a[0, 0] + loss_b[0, 0]) * (1.25 / (_ROWS * _DIM))
    return quantized, vq_loss


# speedup vs baseline: 1.1021x; 1.1021x over previous
import functools

import jax
import jax.numpy as jnp
from jax import lax
from jax.experimental import pallas as pl
from jax.experimental.pallas import tpu as pltpu
from jax.experimental.pallas import tpu_sc as plsc

_N_EMB = 1024
_DIM = 64
_ROWS = 9216
_TILE = 1024
_N_TILES = _ROWS // _TILE

_NW = 32
_B_PER_W = _ROWS // _NW
_CHUNK = 96
_N_CHUNK = _B_PER_W // _CHUNK


def _vq_tc_body(x_ref, emb_ref, idx_ref, loss_ref, emb_c_ref):
    emb = emb_ref[...]
    emb_t = emb.T
    b = jnp.sum(emb_t * emb_t, axis=0, keepdims=True)
    xt_all = x_ref[...].reshape(_ROWS, _DIM)
    loss = jnp.zeros((1, 1), jnp.float32)
    for i in range(_N_TILES):
        xt = lax.slice(xt_all, (i * _TILE, 0), ((i + 1) * _TILE, _DIM))
        a = jnp.sum(xt * xt, axis=1, keepdims=True)
        c2 = lax.dot_general(xt + xt, emb_t, (((1,), (0,)), ((), ())),
                             preferred_element_type=jnp.float32)
        d = (a + b) - c2
        m = jnp.min(d, axis=1, keepdims=True)
        iota = lax.broadcasted_iota(jnp.int32, d.shape, 1)
        cand = jnp.where(d == m, iota, _N_EMB)
        s = [lax.slice(cand, (0, 128 * k), (_TILE, 128 * (k + 1)))
             for k in range(8)]
        p = jnp.minimum(jnp.minimum(jnp.minimum(s[0], s[1]),
                                    jnp.minimum(s[2], s[3])),
                        jnp.minimum(jnp.minimum(s[4], s[5]),
                                    jnp.minimum(s[6], s[7])))
        idx_row = jnp.min(p.T, axis=0, keepdims=True)
        idx_ref[pl.ds(_TILE // 128 * i, _TILE // 128), :] = idx_row.reshape(_TILE // 128, 128)
        loss = loss + jnp.sum(m, keepdims=True)
    loss_ref[...] = loss * (1.25 / (_ROWS * _DIM))

    rr = lax.broadcasted_iota(jnp.int32, (_N_EMB // 2, _N_EMB), 0)
    cc = lax.broadcasted_iota(jnp.int32, (_N_EMB // 2, _N_EMB), 1)
    s_even = (cc == rr + rr).astype(jnp.float32)
    s_odd = (cc == rr + rr + 1).astype(jnp.float32)
    dn = (((1,), (0,)), ((), ()))
    even = lax.dot_general(s_even, emb, dn,
                           preferred_element_type=jnp.float32)
    odd = lax.dot_general(s_odd, emb, dn,
                          preferred_element_type=jnp.float32)
    emb_c_ref[...] = jnp.concatenate([even, odd], axis=1)


def _distances_argmin_loss(x, emb):
    return pl.pallas_call(
        _vq_tc_body,
        in_specs=[
            pl.BlockSpec(x.shape, lambda: (0, 0, 0)),
            pl.BlockSpec((_N_EMB, _DIM), lambda: (0, 0)),
        ],
        out_specs=[
            pl.BlockSpec((_ROWS // 128, 128), lambda: (0, 0)),
            pl.BlockSpec((1, 1), lambda: (0, 0)),
            pl.BlockSpec((_N_EMB // 2, 2 * _DIM), lambda: (0, 0)),
        ],
        out_shape=[
            jax.ShapeDtypeStruct((_ROWS // 128, 128), jnp.int32),
            jax.ShapeDtypeStruct((1, 1), jnp.float32),
            jax.ShapeDtypeStruct((_N_EMB // 2, 2 * _DIM), jnp.float32),
        ],
    )(x, emb)


@functools.cache
def _make_sc_gather():
    @functools.partial(
        pl.kernel,
        mesh=plsc.VectorSubcoreMesh(core_axis_name="c", subcore_axis_name="s"),
        out_type=jax.ShapeDtypeStruct((16, 576, _DIM), jnp.float32),
        scratch_types=[
            pltpu.VMEM((_N_CHUNK, _CHUNK), jnp.int32),
            pltpu.VMEM((_B_PER_W, _DIM), jnp.float32),
            pltpu.SemaphoreType.DMA,
            pltpu.SemaphoreType.DMA,
        ],
        compiler_params=pltpu.CompilerParams(use_tc_tiling_on_sc=False),
    )
    def _sc_gather(table_hbm, idx_hbm, out_hbm, idx_v, rows_v, sem, sem2):
        wid = lax.axis_index("s") * 2 + lax.axis_index("c")
        base = (wid % 2) * _B_PER_W
        slab = wid // 2
        pltpu.sync_copy(idx_hbm.at[wid], idx_v)
        copies = [
            pltpu.async_copy(table_hbm.at[idx_v.at[c]],
                             rows_v.at[pl.ds(c * _CHUNK, _CHUNK)], sem)
            for c in range(_N_CHUNK)
        ]
        writebacks = []
        for c in range(_N_CHUNK):
            copies[c].wait()
            writebacks.append(pltpu.async_copy(
                rows_v.at[pl.ds(c * _CHUNK, _CHUNK)],
                out_hbm.at[slab, pl.ds(base + c * _CHUNK, _CHUNK)], sem2))
        for wb in writebacks:
            wb.wait()

    return _sc_gather


def kernel(x, emb_weight):
    idx, loss, emb_c = _distances_argmin_loss(x, emb_weight)
    idx3 = idx.reshape(_NW, _N_CHUNK, _CHUNK)
    quantized = _make_sc_gather()(emb_c.reshape(_N_EMB, _DIM), idx3)
    return quantized, loss.reshape(())


# speedup vs baseline: 1.1099x; 1.0071x over previous
import functools

import jax
import jax.numpy as jnp
from jax import lax
from jax.experimental import pallas as pl
from jax.experimental.pallas import tpu as pltpu
from jax.experimental.pallas import tpu_sc as plsc

_N_EMB = 1024
_DIM = 64
_ROWS = 9216
_TILE = 1024
_N_TILES = _ROWS // _TILE

_NW = 32
_B_PER_W = _ROWS // _NW
_CHUNK = 96
_N_CHUNK = _B_PER_W // _CHUNK


def _vq_tc_body(x_ref, emb_ref, idx_ref, loss_ref, emb_c_ref):
    emb = emb_ref[...]
    emb_t = emb.T
    b = jnp.sum(emb_t * emb_t, axis=0, keepdims=True)
    xt_all = x_ref[...].reshape(_ROWS, _DIM)
    loss = jnp.zeros((1, 1), jnp.float32)
    for i in range(_N_TILES):
        xt = lax.slice(xt_all, (i * _TILE, 0), ((i + 1) * _TILE, _DIM))
        a = jnp.sum(xt * xt, axis=1, keepdims=True)
        c2 = lax.dot_general(xt + xt, emb_t, (((1,), (0,)), ((), ())),
                             preferred_element_type=jnp.float32)
        d = (a + b) - c2
        m = jnp.min(d, axis=1, keepdims=True)
        iota = lax.broadcasted_iota(jnp.int32, d.shape, 1)
        cand = jnp.where(d == m, iota, _N_EMB)
        s = [lax.slice(cand, (0, 128 * k), (_TILE, 128 * (k + 1)))
             for k in range(8)]
        p = jnp.minimum(jnp.minimum(jnp.minimum(s[0], s[1]),
                                    jnp.minimum(s[2], s[3])),
                        jnp.minimum(jnp.minimum(s[4], s[5]),
                                    jnp.minimum(s[6], s[7])))
        idx_row = jnp.min(p.T, axis=0, keepdims=True)
        idx_ref[pl.ds(_TILE // 128 * i, _TILE // 128), :] = idx_row.reshape(_TILE // 128, 128)
        loss = loss + jnp.sum(m, keepdims=True)
    loss_ref[...] = loss * (1.25 / (_ROWS * _DIM))

    rr = lax.broadcasted_iota(jnp.int32, (_N_EMB // 2, _N_EMB), 0)
    cc = lax.broadcasted_iota(jnp.int32, (_N_EMB // 2, _N_EMB), 1)
    s_even = (cc == rr + rr).astype(jnp.float32)
    s_odd = (cc == rr + rr + 1).astype(jnp.float32)
    dn = (((1,), (0,)), ((), ()))
    even = lax.dot_general(s_even, emb, dn,
                           preferred_element_type=jnp.float32)
    odd = lax.dot_general(s_odd, emb, dn,
                          preferred_element_type=jnp.float32)
    emb_c_ref[...] = jnp.concatenate([even, odd], axis=1)


def _distances_argmin_loss(x, emb):
    return pl.pallas_call(
        _vq_tc_body,
        in_specs=[
            pl.BlockSpec(x.shape, lambda: (0, 0, 0)),
            pl.BlockSpec((_N_EMB, _DIM), lambda: (0, 0)),
        ],
        out_specs=[
            pl.BlockSpec((_ROWS // 128, 128), lambda: (0, 0)),
            pl.BlockSpec((1, 1), lambda: (0, 0)),
            pl.BlockSpec((_N_EMB // 2, 2 * _DIM), lambda: (0, 0)),
        ],
        out_shape=[
            jax.ShapeDtypeStruct((_ROWS // 128, 128), jnp.int32),
            jax.ShapeDtypeStruct((1, 1), jnp.float32),
            jax.ShapeDtypeStruct((_N_EMB // 2, 2 * _DIM), jnp.float32),
        ],
    )(x, emb)


@functools.cache
def _make_sc_gather():
    @functools.partial(
        pl.kernel,
        mesh=plsc.VectorSubcoreMesh(core_axis_name="c", subcore_axis_name="s"),
        out_type=jax.ShapeDtypeStruct((16, 576, _DIM), jnp.float32),
        scratch_types=[
            pltpu.VMEM((_N_CHUNK, _CHUNK), jnp.int32),
            pltpu.VMEM((_B_PER_W, _DIM), jnp.float32),
            pltpu.SemaphoreType.DMA,
        ],
        compiler_params=pltpu.CompilerParams(use_tc_tiling_on_sc=False),
    )
    def _sc_gather(table_hbm, idx_hbm, out_hbm, idx_v, rows_v, sem):
        wid = lax.axis_index("s") * 2 + lax.axis_index("c")
        pltpu.sync_copy(idx_hbm.at[wid], idx_v)
        copies = [
            pltpu.async_copy(table_hbm.at[idx_v.at[c]],
                             rows_v.at[pl.ds(c * _CHUNK, _CHUNK)], sem)
            for c in range(_N_CHUNK)
        ]
        for cp in copies:
            cp.wait()
        pltpu.sync_copy(
            rows_v,
            out_hbm.at[wid // 2, pl.ds((wid % 2) * _B_PER_W, _B_PER_W)])

    return _sc_gather


def kernel(x, emb_weight):
    idx, loss, emb_c = _distances_argmin_loss(x, emb_weight)
    idx3 = idx.reshape(_NW, _N_CHUNK, _CHUNK)
    quantized = _make_sc_gather()(emb_c.reshape(_N_EMB, _DIM), idx3)
    return quantized, loss.reshape(())
